# Initial kernel scaffold; baseline (speedup 1.0000x reference)
import jax
import jax.numpy as jnp
from jax.experimental import pallas as pl


def kernel(x, edge_index, W1l, b1, W1r, W2l, b2, W2r):
    raise NotImplementedError("write your pallas kernel here")


# speedup vs baseline: 2.3062x; 2.3062x over previous
import functools

import jax
import jax.numpy as jnp
from jax import lax
from jax.experimental import pallas as pl
from jax.experimental.pallas import tpu as pltpu
from jax.experimental.pallas import tpu_sc as plsc

NW = 32
CAP = 16384
CE = 4000
CG = 128
CG2 = 256
LPAD = CAP + 512


def _sc_layer1(n_nodes, n_edges, feat, npt):
    npad = NW * npt
    nfv = feat // 16
    mesh = plsc.VectorSubcoreMesh(core_axis_name="c", subcore_axis_name="s")

    @functools.partial(
        pl.kernel,
        out_type=[
            jax.ShapeDtypeStruct((npad, feat), jnp.float32),
            jax.ShapeDtypeStruct((NW, LPAD), jnp.int32),
            jax.ShapeDtypeStruct((NW, LPAD), jnp.int32),
            jax.ShapeDtypeStruct((NW, 16), jnp.int32),
        ],
        mesh=mesh,
        compiler_params=pltpu.CompilerParams(needs_layout_passes=False),
        scratch_types=[
            pltpu.VMEM((npt + 1, feat), jnp.float32),
            pltpu.VMEM((LPAD,), jnp.int32),
            pltpu.VMEM((LPAD,), jnp.int32),
            pltpu.VMEM((CE,), jnp.int32),
            pltpu.VMEM((CE,), jnp.int32),
            pltpu.VMEM((CG, feat), jnp.float32),
            pltpu.VMEM((16,), jnp.int32),
            pltpu.SemaphoreType.DMA,
        ],
    )
    def ka(x_hbm, esrc_hbm, edst_hbm, agg_hbm, srcl_hbm, dstl_hbm, cnt_hbm,
           acc, srcl, dstl, srch, dsth, rows, cstg, sem):
        w = lax.axis_index("s") * 2 + lax.axis_index("c")
        n0 = w * npt

        minf = jnp.full((16,), -jnp.inf, jnp.float32)
        zero16 = jnp.zeros((16,), jnp.int32)

        def initacc(r, carry):
            for j in range(nfv):
                acc[r, pl.ds(16 * j, 16)] = minf
            return carry
        lax.fori_loop(0, npt + 1, initacc, 0)

        def initsrc(i, carry):
            srcl[pl.ds(16 * i, 16)] = zero16
            return carry
        lax.fori_loop(0, LPAD // 16, initsrc, 0)

        def chunk_body(c, ptr):
            e0 = c * CE
            pltpu.sync_copy(esrc_hbm.at[pl.ds(e0, CE)], srch)
            pltpu.sync_copy(edst_hbm.at[pl.ds(e0, CE)], dsth)

            iota = jnp.arange(16, dtype=jnp.int32)

            def vec_body(i, p):
                dv = dsth[pl.ds(16 * i, 16)]
                sv = srch[pl.ds(16 * i, 16)]
                off = dv - n0
                m = (off >= 0) & (off < npt)
                s = jnp.where(m, jnp.int32(1), jnp.int32(0))
                for k in (1, 2, 4, 8):
                    g = jnp.take_along_axis(s, jnp.maximum(iota - k, 0),
                                            axis=0, mode="promise_in_bounds")
                    s = s + jnp.where(iota >= k, g, jnp.int32(0))
                pos = p + s - 1
                m2 = m & (pos < CAP)
                plsc.store_scatter(srcl, [pos], sv, mask=m2)
                plsc.store_scatter(dstl, [pos], off, mask=m2)
                return p + s[15]

            return lax.fori_loop(0, CE // 16, vec_body, ptr)

        ptr = lax.fori_loop(0, n_edges // CE, chunk_body, jnp.int32(0))
        cnt = jnp.minimum(ptr, jnp.int32(CAP))

        srcl[pl.ds(cnt, 16)] = zero16
        dstl[pl.ds(cnt, 16)] = jnp.full((16,), npt, jnp.int32)
        cntp = ((cnt + 15) // 16) * 16

        nck = (cntp + CG - 1) // CG

        def gchunk(c, carry):
            base = c * CG
            pltpu.async_copy(
                x_hbm.at[srcl.at[pl.ds(base, CG)]], rows, sem).wait()
            ng = jnp.minimum(jnp.int32(CG // 16), (cntp - base) // 16)

            def gbody(g, c2):
                dv = dstl[pl.ds(base + 16 * g, 16)]
                for lane in range(16):
                    d = dv[lane]
                    r = 16 * g + lane
                    for j in range(nfv):
                        sl = pl.ds(16 * j, 16)
                        acc[d, sl] = jnp.maximum(acc[d, sl], rows[r, sl])
                return c2
            lax.fori_loop(0, ng, gbody, 0)
            return carry
        lax.fori_loop(0, nck, gchunk, 0)

        pltpu.sync_copy(acc.at[pl.ds(0, npt)], agg_hbm.at[pl.ds(n0, npt)])
        pltpu.sync_copy(srcl, srcl_hbm.at[w])
        pltpu.sync_copy(dstl, dstl_hbm.at[w])
        cstg[...] = jnp.full((16,), cntp, jnp.int32)
        pltpu.sync_copy(cstg, cnt_hbm.at[w])

    return ka


def _sc_layer2(n_nodes, feat, fpad, npt):
    npad = NW * npt
    nfv = feat // 16
    mesh = plsc.VectorSubcoreMesh(core_axis_name="c", subcore_axis_name="s")

    @functools.partial(
        pl.kernel,
        out_type=jax.ShapeDtypeStruct((npad, feat), jnp.float32),
        mesh=mesh,
        compiler_params=pltpu.CompilerParams(needs_layout_passes=False),
        scratch_types=[
            pltpu.VMEM((npt + 1, feat), jnp.float32),
            pltpu.VMEM((LPAD,), jnp.int32),
            pltpu.VMEM((LPAD,), jnp.int32),
            pltpu.VMEM((CG2, fpad), jnp.float32),
            pltpu.VMEM((16,), jnp.int32),
            pltpu.SemaphoreType.DMA,
        ],
    )
    def kc(h_hbm, srcl_hbm, dstl_hbm, cnt_hbm, agg_hbm,
           acc, srcl, dstl, rows, cstg, sem):
        w = lax.axis_index("s") * 2 + lax.axis_index("c")
        n0 = w * npt

        pltpu.sync_copy(cnt_hbm.at[w], cstg)
        pltpu.sync_copy(srcl_hbm.at[w], srcl)
        pltpu.sync_copy(dstl_hbm.at[w], dstl)
        cntp = cstg[pl.ds(0, 16)][0]

        minf = jnp.full((16,), -jnp.inf, jnp.float32)

        def initacc(r, carry):
            for j in range(nfv):
                acc[r, pl.ds(16 * j, 16)] = minf
            return carry
        lax.fori_loop(0, npt + 1, initacc, 0)

        nck = (cntp + CG2 - 1) // CG2

        def gchunk(c, carry):
            base = c * CG2
            pltpu.async_copy(
                h_hbm.at[srcl.at[pl.ds(base, CG2)]], rows, sem).wait()
            ng = jnp.minimum(jnp.int32(CG2 // 16), (cntp - base) // 16)

            def gbody(g, c2):
                dv = dstl[pl.ds(base + 16 * g, 16)]
                for lane in range(16):
                    d = dv[lane]
                    r = 16 * g + lane
                    for j in range(nfv):
                        sl = pl.ds(16 * j, 16)
                        acc[d, sl] = jnp.maximum(acc[d, sl], rows[r, sl])
                return c2
            lax.fori_loop(0, ng, gbody, 0)
            return carry
        lax.fori_loop(0, nck, gchunk, 0)

        pltpu.sync_copy(acc.at[pl.ds(0, npt)], agg_hbm.at[pl.ds(n0, npt)])

    return kc


def _tc_dense(m, ka, kx, n, relu, bm, out_pad=None):
    nw = out_pad or n

    def body(a_ref, x_ref, wl_ref, wr_ref, b_ref, o_ref):
        a = a_ref[...]
        a = jnp.where(a == -jnp.inf, jnp.float32(0.0), a)
        r = (jnp.dot(a, wl_ref[...], preferred_element_type=jnp.float32)
             + jnp.dot(x_ref[...], wr_ref[...],
                       preferred_element_type=jnp.float32)
             + b_ref[...])
        if relu:
            r = jnp.maximum(r, jnp.float32(0.0))
        if out_pad is None:
            o_ref[...] = r
        else:
            o_ref[:, pl.ds(0, n)] = r
            o_ref[:, pl.ds(n, out_pad - n)] = jnp.zeros(
                (bm, out_pad - n), jnp.float32)

    grid = (m // bm,)
    return pl.pallas_call(
        body,
        grid=grid,
        in_specs=[
            pl.BlockSpec((bm, ka), lambda i: (i, 0)),
            pl.BlockSpec((bm, kx), lambda i: (i, 0)),
            pl.BlockSpec((ka, n), lambda i: (0, 0)),
            pl.BlockSpec((kx, n), lambda i: (0, 0)),
            pl.BlockSpec((1, n), lambda i: (0, 0)),
        ],
        out_specs=pl.BlockSpec((bm, nw), lambda i: (i, 0)),
        out_shape=jax.ShapeDtypeStruct((m, nw), jnp.float32),
    )


def kernel(x, edge_index, W1l, b1, W1r, W2l, b2, W2r):
    n, d = x.shape
    e = edge_index.shape[1]
    h_dim = W1l.shape[1]
    c_dim = W2l.shape[1]
    npt = (-(-n // NW) + 7) // 8 * 8

    esrc = edge_index[0]
    edst = edge_index[1]
    agg1p, srcl, dstl, cnts = _sc_layer1(n, e, d, npt)(x, esrc, edst)
    agg1 = agg1p[:n]

    hp = _tc_dense(n, d, d, h_dim, True, 2000, out_pad=128)(
        agg1, x, W1l, W1r, b1.reshape(1, -1))

    agg2p = _sc_layer2(n, h_dim, 128, npt)(hp, srcl, dstl, cnts)
    agg2 = agg2p[:n]

    cpad = 8
    w2l = jnp.zeros((h_dim, cpad), jnp.float32).at[:, :c_dim].set(W2l)
    w2rp = jnp.zeros((128, cpad), jnp.float32).at[:h_dim, :c_dim].set(W2r)
    b2p = jnp.zeros((1, cpad), jnp.float32).at[0, :c_dim].set(b2)

    out = _tc_dense(n, h_dim, 128, cpad, False, 2000)(
        agg2, hp, w2l, w2rp, b2p)
    return out[:, :c_dim]


# speedup vs baseline: 3.4763x; 1.5073x over previous
import functools

import jax
import jax.numpy as jnp
from jax import lax
from jax.experimental import pallas as pl
from jax.experimental.pallas import tpu as pltpu
from jax.experimental.pallas import tpu_sc as plsc

NW = 32
CAP = 12800
CE = 10000
CG = 128
CG2 = 128
LPAD = CAP + 512


def _sc_layer1(n_nodes, n_edges, feat, npt):
    npad = NW * npt
    nfv = feat // 16
    mesh = plsc.VectorSubcoreMesh(core_axis_name="c", subcore_axis_name="s")

    @functools.partial(
        pl.kernel,
        out_type=[
            jax.ShapeDtypeStruct((npad, feat), jnp.float32),
            jax.ShapeDtypeStruct((NW, LPAD), jnp.int32),
            jax.ShapeDtypeStruct((NW, LPAD), jnp.int32),
            jax.ShapeDtypeStruct((NW, 16), jnp.int32),
        ],
        mesh=mesh,
        compiler_params=pltpu.CompilerParams(needs_layout_passes=False),
        scratch_types=[
            pltpu.VMEM((npt + 1, feat), jnp.float32),
            pltpu.VMEM((LPAD,), jnp.int32),
            pltpu.VMEM((LPAD,), jnp.int32),
            pltpu.VMEM((CE,), jnp.int32),
            pltpu.VMEM((CE,), jnp.int32),
            pltpu.VMEM((CG, feat), jnp.float32),
            pltpu.VMEM((CG, feat), jnp.float32),
            pltpu.VMEM((16,), jnp.int32),
            pltpu.SemaphoreType.DMA,
            pltpu.SemaphoreType.DMA,
        ],
    )
    def ka(x_hbm, esrc_hbm, edst_hbm, agg_hbm, srcl_hbm, dstl_hbm, cnt_hbm,
           acc, srcl, dstl, srch, dsth, rows0, rows1, cstg, sem0, sem1):
        w = lax.axis_index("s") * 2 + lax.axis_index("c")
        n0 = w * npt

        minf = jnp.full((16,), -jnp.inf, jnp.float32)
        zero16 = jnp.zeros((16,), jnp.int32)

        def initacc(r, carry):
            for j in range(nfv):
                acc[r, pl.ds(16 * j, 16)] = minf
            return carry
        lax.fori_loop(0, npt + 1, initacc, 0)

        def initsrc(i, carry):
            srcl[pl.ds(16 * i, 16)] = zero16
            return carry
        lax.fori_loop(0, LPAD // 16, initsrc, 0)

        def chunk_body(c, ptr):
            e0 = c * CE
            pltpu.sync_copy(esrc_hbm.at[pl.ds(e0, CE)], srch)
            pltpu.sync_copy(edst_hbm.at[pl.ds(e0, CE)], dsth)

            def vec_body(i, p):
                dv = dsth[pl.ds(16 * i, 16)]
                sv = srch[pl.ds(16 * i, 16)]
                off = dv - n0
                m = (off >= 0) & (off < npt)
                plsc.store_compressed(srcl.at[pl.ds(p, 16)], sv, mask=m)
                plsc.store_compressed(dstl.at[pl.ds(p, 16)], off, mask=m)
                nm = plsc.all_reduce_population_count(m)[0]
                return jnp.minimum(p + nm, jnp.int32(CAP))

            return lax.fori_loop(0, CE // 16, vec_body, ptr)

        ptr = lax.fori_loop(0, n_edges // CE, chunk_body, jnp.int32(0))
        cnt = jnp.minimum(ptr, jnp.int32(CAP))

        srcl[pl.ds(cnt, 16)] = zero16
        dstl[pl.ds(cnt, 16)] = jnp.full((16,), npt, jnp.int32)
        cntp = ((cnt + 15) // 16) * 16

        nck = (cntp + CG - 1) // CG
        bufs = (rows0, rows1)
        sems = (sem0, sem1)

        @pl.when(nck > 0)
        def _():
            pltpu.async_copy(x_hbm.at[srcl.at[pl.ds(0, CG)]], rows0, sem0)

        def gpair(t, carry):
            for b in range(2):
                c = 2 * t + b
                nb = (b + 1) % 2

                @pl.when(c < nck)
                def _():
                    base = c * CG

                    @pl.when(c + 1 < nck)
                    def _():
                        pltpu.async_copy(
                            x_hbm.at[srcl.at[pl.ds((c + 1) * CG, CG)]],
                            bufs[nb], sems[nb])

                    pltpu.make_async_copy(
                        x_hbm.at[srcl.at[pl.ds(base, CG)]],
                        bufs[b], sems[b]).wait()
                    rows = bufs[b]
                    ng = jnp.minimum(jnp.int32(CG // 16),
                                     (cntp - base) // 16)

                    def gbody(g, c2):
                        dv = dstl[pl.ds(base + 16 * g, 16)]
                        for lane in range(16):
                            d = dv[lane]
                            r = 16 * g + lane
                            for j in range(nfv):
                                sl = pl.ds(16 * j, 16)
                                acc[d, sl] = jnp.maximum(
                                    acc[d, sl], rows[r, sl])
                        return c2
                    lax.fori_loop(0, ng, gbody, 0)
            return carry
        lax.fori_loop(0, (nck + 1) // 2, gpair, 0)

        pltpu.sync_copy(acc.at[pl.ds(0, npt)], agg_hbm.at[pl.ds(n0, npt)])
        pltpu.sync_copy(srcl, srcl_hbm.at[w])
        pltpu.sync_copy(dstl, dstl_hbm.at[w])
        cstg[...] = jnp.full((16,), cntp, jnp.int32)
        pltpu.sync_copy(cstg, cnt_hbm.at[w])

    return ka


def _sc_layer2(n_nodes, feat, fpad, npt):
    npad = NW * npt
    nfv = feat // 16
    mesh = plsc.VectorSubcoreMesh(core_axis_name="c", subcore_axis_name="s")

    @functools.partial(
        pl.kernel,
        out_type=jax.ShapeDtypeStruct((npad, feat), jnp.float32),
        mesh=mesh,
        compiler_params=pltpu.CompilerParams(needs_layout_passes=False),
        scratch_types=[
            pltpu.VMEM((npt + 1, feat), jnp.float32),
            pltpu.VMEM((LPAD,), jnp.int32),
            pltpu.VMEM((LPAD,), jnp.int32),
            pltpu.VMEM((CG2, fpad), jnp.float32),
            pltpu.VMEM((CG2, fpad), jnp.float32),
            pltpu.VMEM((16,), jnp.int32),
            pltpu.SemaphoreType.DMA,
            pltpu.SemaphoreType.DMA,
        ],
    )
    def kc(h_hbm, srcl_hbm, dstl_hbm, cnt_hbm, agg_hbm,
           acc, srcl, dstl, rows0, rows1, cstg, sem0, sem1):
        w = lax.axis_index("s") * 2 + lax.axis_index("c")
        n0 = w * npt

        pltpu.sync_copy(cnt_hbm.at[w], cstg)
        pltpu.sync_copy(srcl_hbm.at[w], srcl)
        pltpu.sync_copy(dstl_hbm.at[w], dstl)
        cntp = cstg[pl.ds(0, 16)][0]

        minf = jnp.full((16,), -jnp.inf, jnp.float32)

        def initacc(r, carry):
            for j in range(nfv):
                acc[r, pl.ds(16 * j, 16)] = minf
            return carry
        lax.fori_loop(0, npt + 1, initacc, 0)

        nck = (cntp + CG2 - 1) // CG2
        bufs = (rows0, rows1)
        sems = (sem0, sem1)

        @pl.when(nck > 0)
        def _():
            pltpu.async_copy(h_hbm.at[srcl.at[pl.ds(0, CG2)]], rows0, sem0)

        def gpair(t, carry):
            for b in range(2):
                c = 2 * t + b
                nb = (b + 1) % 2

                @pl.when(c < nck)
                def _():
                    base = c * CG2

                    @pl.when(c + 1 < nck)
                    def _():
                        pltpu.async_copy(
                            h_hbm.at[srcl.at[pl.ds((c + 1) * CG2, CG2)]],
                            bufs[nb], sems[nb])

                    pltpu.make_async_copy(
                        h_hbm.at[srcl.at[pl.ds(base, CG2)]],
                        bufs[b], sems[b]).wait()
                    rows = bufs[b]
                    ng = jnp.minimum(jnp.int32(CG2 // 16),
                                     (cntp - base) // 16)

                    def gbody(g, c2):
                        dv = dstl[pl.ds(base + 16 * g, 16)]
                        for lane in range(16):
                            d = dv[lane]
                            r = 16 * g + lane
                            for j in range(nfv):
                                sl = pl.ds(16 * j, 16)
                                acc[d, sl] = jnp.maximum(
                                    acc[d, sl], rows[r, sl])
                        return c2
                    lax.fori_loop(0, ng, gbody, 0)
            return carry
        lax.fori_loop(0, (nck + 1) // 2, gpair, 0)

        pltpu.sync_copy(acc.at[pl.ds(0, npt)], agg_hbm.at[pl.ds(n0, npt)])

    return kc


def _tc_dense(m, ka, kx, n, relu, bm, out_pad=None):
    nw = out_pad or n

    def body(a_ref, x_ref, wl_ref, wr_ref, b_ref, o_ref):
        a = a_ref[...]
        a = jnp.where(a == -jnp.inf, jnp.float32(0.0), a)
        r = (jnp.dot(a, wl_ref[...], preferred_element_type=jnp.float32)
             + jnp.dot(x_ref[...], wr_ref[...],
                       preferred_element_type=jnp.float32)
             + b_ref[...])
        if relu:
            r = jnp.maximum(r, jnp.float32(0.0))
        if out_pad is None:
            o_ref[...] = r
        else:
            o_ref[:, pl.ds(0, n)] = r
            o_ref[:, pl.ds(n, out_pad - n)] = jnp.zeros(
                (bm, out_pad - n), jnp.float32)

    grid = (m // bm,)
    return pl.pallas_call(
        body,
        grid=grid,
        in_specs=[
            pl.BlockSpec((bm, ka), lambda i: (i, 0)),
            pl.BlockSpec((bm, kx), lambda i: (i, 0)),
            pl.BlockSpec((ka, n), lambda i: (0, 0)),
            pl.BlockSpec((kx, n), lambda i: (0, 0)),
            pl.BlockSpec((1, n), lambda i: (0, 0)),
        ],
        out_specs=pl.BlockSpec((bm, nw), lambda i: (i, 0)),
        out_shape=jax.ShapeDtypeStruct((m, nw), jnp.float32),
    )


def kernel(x, edge_index, W1l, b1, W1r, W2l, b2, W2r):
    n, d = x.shape
    e = edge_index.shape[1]
    h_dim = W1l.shape[1]
    c_dim = W2l.shape[1]
    npt = (-(-n // NW) + 7) // 8 * 8

    esrc = edge_index[0]
    edst = edge_index[1]
    agg1p, srcl, dstl, cnts = _sc_layer1(n, e, d, npt)(x, esrc, edst)
    agg1 = agg1p[:n]

    hp = _tc_dense(n, d, d, h_dim, True, 2000, out_pad=128)(
        agg1, x, W1l, W1r, b1.reshape(1, -1))

    agg2p = _sc_layer2(n, h_dim, 128, npt)(hp, srcl, dstl, cnts)
    agg2 = agg2p[:n]

    cpad = 8
    w2l = jnp.zeros((h_dim, cpad), jnp.float32).at[:, :c_dim].set(W2l)
    w2rp = jnp.zeros((128, cpad), jnp.float32).at[:h_dim, :c_dim].set(W2r)
    b2p = jnp.zeros((1, cpad), jnp.float32).at[0, :c_dim].set(b2)

    out = _tc_dense(n, h_dim, 128, cpad, False, 2000)(
        agg2, hp, w2l, w2rp, b2p)
    return out[:, :c_dim]


# speedup vs baseline: 3.7575x; 1.0809x over previous
import functools

import jax
import jax.numpy as jnp
from jax import lax
from jax.experimental import pallas as pl
from jax.experimental.pallas import tpu as pltpu
from jax.experimental.pallas import tpu_sc as plsc

NW = 32
CAP = 12800
CE = 8000
CG = 96
CG2 = 128
LPAD = CAP + 512


def _sc_layer1(n_nodes, n_edges, feat, npt):
    npad = NW * npt
    nfv = feat // 16
    mesh = plsc.VectorSubcoreMesh(core_axis_name="c", subcore_axis_name="s")

    @functools.partial(
        pl.kernel,
        out_type=[
            jax.ShapeDtypeStruct((npad, feat), jnp.float32),
            jax.ShapeDtypeStruct((NW, LPAD), jnp.int32),
            jax.ShapeDtypeStruct((NW, LPAD), jnp.int32),
            jax.ShapeDtypeStruct((NW, 16), jnp.int32),
        ],
        mesh=mesh,
        compiler_params=pltpu.CompilerParams(needs_layout_passes=False),
        scratch_types=[
            pltpu.VMEM((npt + 1, feat), jnp.float32),
            pltpu.VMEM((LPAD,), jnp.int32),
            pltpu.VMEM((LPAD,), jnp.int32),
            pltpu.VMEM((CE,), jnp.int32),
            pltpu.VMEM((CE,), jnp.int32),
            pltpu.VMEM((CE,), jnp.int32),
            pltpu.VMEM((CE,), jnp.int32),
            pltpu.VMEM((CG, feat), jnp.float32),
            pltpu.VMEM((CG, feat), jnp.float32),
            pltpu.VMEM((16,), jnp.int32),
            pltpu.SemaphoreType.DMA,
            pltpu.SemaphoreType.DMA,
            pltpu.SemaphoreType.DMA,
            pltpu.SemaphoreType.DMA,
        ],
    )
    def ka(x_hbm, esrc_hbm, edst_hbm, agg_hbm, srcl_hbm, dstl_hbm, cnt_hbm,
           acc, srcl, dstl, srch0, srch1, dsth0, dsth1, rows0, rows1, cstg,
           sem0, sem1, sem2, sem3):
        w = lax.axis_index("s") * 2 + lax.axis_index("c")
        n0 = w * npt

        minf = jnp.full((16,), -jnp.inf, jnp.float32)
        zero16 = jnp.zeros((16,), jnp.int32)

        def initacc(r, carry):
            for j in range(nfv):
                acc[r, pl.ds(16 * j, 16)] = minf
            return carry
        lax.fori_loop(0, npt + 1, initacc, 0)

        def initsrc(i, carry):
            srcl[pl.ds(16 * i, 16)] = zero16
            return carry
        lax.fori_loop(0, LPAD // 16, initsrc, 0)

        nch = n_edges // CE
        sbufs = (srch0, srch1)
        dbufs = (dsth0, dsth1)
        esems = (sem0, sem1)
        dsems = (sem2, sem3)

        pltpu.async_copy(esrc_hbm.at[pl.ds(0, CE)], srch0, sem0)
        pltpu.async_copy(edst_hbm.at[pl.ds(0, CE)], dsth0, sem2)

        def cpair(t, ptr):
            for b in range(2):
                c = 2 * t + b
                nb = 1 - b

                @pl.when(c + 1 < nch)
                def _():
                    e1 = (c + 1) * CE
                    pltpu.async_copy(
                        esrc_hbm.at[pl.ds(e1, CE)], sbufs[nb], esems[nb])
                    pltpu.async_copy(
                        edst_hbm.at[pl.ds(e1, CE)], dbufs[nb], dsems[nb])

                e0 = c * CE
                pltpu.make_async_copy(
                    esrc_hbm.at[pl.ds(e0, CE)], sbufs[b], esems[b]).wait()
                pltpu.make_async_copy(
                    edst_hbm.at[pl.ds(e0, CE)], dbufs[b], dsems[b]).wait()
                srch = sbufs[b]
                dsth = dbufs[b]

                def vec_body(i, p):
                    dv0 = dsth[pl.ds(32 * i, 16)]
                    sv0 = srch[pl.ds(32 * i, 16)]
                    dv1 = dsth[pl.ds(32 * i + 16, 16)]
                    sv1 = srch[pl.ds(32 * i + 16, 16)]
                    off0 = dv0 - n0
                    m0 = (off0 >= 0) & (off0 < npt)
                    off1 = dv1 - n0
                    m1 = (off1 >= 0) & (off1 < npt)
                    c0 = plsc.all_reduce_population_count(m0)[0]
                    c1 = plsc.all_reduce_population_count(m1)[0]
                    plsc.store_compressed(
                        srcl.at[pl.ds(p, 16)], sv0, mask=m0)
                    plsc.store_compressed(
                        dstl.at[pl.ds(p, 16)], off0, mask=m0)
                    p1 = jnp.minimum(p + c0, jnp.int32(CAP))
                    plsc.store_compressed(
                        srcl.at[pl.ds(p1, 16)], sv1, mask=m1)
                    plsc.store_compressed(
                        dstl.at[pl.ds(p1, 16)], off1, mask=m1)
                    return jnp.minimum(p1 + c1, jnp.int32(CAP))

                ptr = lax.fori_loop(0, CE // 32, vec_body, ptr)
            return ptr

        ptr = lax.fori_loop(0, nch // 2, cpair, jnp.int32(0))
        cnt = ptr

        srcl[pl.ds(cnt, 16)] = zero16
        dstl[pl.ds(cnt, 16)] = jnp.full((16,), npt, jnp.int32)
        cntp = ((cnt + 15) // 16) * 16

        nck = (cntp + CG - 1) // CG
        bufs = (rows0, rows1)
        sems = (sem0, sem1)

        @pl.when(nck > 0)
        def _():
            pltpu.async_copy(x_hbm.at[srcl.at[pl.ds(0, CG)]], rows0, sem0)

        def gpair(t, carry):
            for b in range(2):
                c = 2 * t + b
                nb = (b + 1) % 2

                @pl.when(c < nck)
                def _():
                    base = c * CG

                    @pl.when(c + 1 < nck)
                    def _():
                        pltpu.async_copy(
                            x_hbm.at[srcl.at[pl.ds((c + 1) * CG, CG)]],
                            bufs[nb], sems[nb])

                    pltpu.make_async_copy(
                        x_hbm.at[srcl.at[pl.ds(base, CG)]],
                        bufs[b], sems[b]).wait()
                    rows = bufs[b]
                    ng = jnp.minimum(jnp.int32(CG // 16),
                                     (cntp - base) // 16)

                    def gbody(g, c2):
                        dv = dstl[pl.ds(base + 16 * g, 16)]
                        for lane in range(16):
                            d = dv[lane]
                            r = 16 * g + lane
                            for j in range(nfv):
                                sl = pl.ds(16 * j, 16)
                                acc[d, sl] = jnp.maximum(
                                    acc[d, sl], rows[r, sl])
                        return c2
                    lax.fori_loop(0, ng, gbody, 0)
            return carry
        lax.fori_loop(0, (nck + 1) // 2, gpair, 0)

        pltpu.sync_copy(acc.at[pl.ds(0, npt)], agg_hbm.at[pl.ds(n0, npt)])
        pltpu.sync_copy(srcl, srcl_hbm.at[w])
        pltpu.sync_copy(dstl, dstl_hbm.at[w])
        cstg[...] = jnp.full((16,), cntp, jnp.int32)
        pltpu.sync_copy(cstg, cnt_hbm.at[w])

    return ka


def _sc_layer2(n_nodes, feat, fpad, npt):
    npad = NW * npt
    nfv = feat // 16
    mesh = plsc.VectorSubcoreMesh(core_axis_name="c", subcore_axis_name="s")

    @functools.partial(
        pl.kernel,
        out_type=jax.ShapeDtypeStruct((npad, feat), jnp.float32),
        mesh=mesh,
        compiler_params=pltpu.CompilerParams(needs_layout_passes=False),
        scratch_types=[
            pltpu.VMEM((npt + 1, feat), jnp.float32),
            pltpu.VMEM((LPAD,), jnp.int32),
            pltpu.VMEM((LPAD,), jnp.int32),
            pltpu.VMEM((CG2, fpad), jnp.float32),
            pltpu.VMEM((CG2, fpad), jnp.float32),
            pltpu.VMEM((16,), jnp.int32),
            pltpu.SemaphoreType.DMA,
            pltpu.SemaphoreType.DMA,
        ],
    )
    def kc(h_hbm, srcl_hbm, dstl_hbm, cnt_hbm, agg_hbm,
           acc, srcl, dstl, rows0, rows1, cstg, sem0, sem1):
        w = lax.axis_index("s") * 2 + lax.axis_index("c")
        n0 = w * npt

        pltpu.sync_copy(cnt_hbm.at[w], cstg)
        pltpu.sync_copy(srcl_hbm.at[w], srcl)
        pltpu.sync_copy(dstl_hbm.at[w], dstl)
        cntp = cstg[pl.ds(0, 16)][0]

        minf = jnp.full((16,), -jnp.inf, jnp.float32)

        def initacc(r, carry):
            for j in range(nfv):
                acc[r, pl.ds(16 * j, 16)] = minf
            return carry
        lax.fori_loop(0, npt + 1, initacc, 0)

        nck = (cntp + CG2 - 1) // CG2
        bufs = (rows0, rows1)
        sems = (sem0, sem1)

        @pl.when(nck > 0)
        def _():
            pltpu.async_copy(h_hbm.at[srcl.at[pl.ds(0, CG2)]], rows0, sem0)

        def gpair(t, carry):
            for b in range(2):
                c = 2 * t + b
                nb = (b + 1) % 2

                @pl.when(c < nck)
                def _():
                    base = c * CG2

                    @pl.when(c + 1 < nck)
                    def _():
                        pltpu.async_copy(
                            h_hbm.at[srcl.at[pl.ds((c + 1) * CG2, CG2)]],
                            bufs[nb], sems[nb])

                    pltpu.make_async_copy(
                        h_hbm.at[srcl.at[pl.ds(base, CG2)]],
                        bufs[b], sems[b]).wait()
                    rows = bufs[b]
                    ng = jnp.minimum(jnp.int32(CG2 // 16),
                                     (cntp - base) // 16)

                    def gbody(g, c2):
                        dv = dstl[pl.ds(base + 16 * g, 16)]
                        for lane in range(16):
                            d = dv[lane]
                            r = 16 * g + lane
                            for j in range(nfv):
                                sl = pl.ds(16 * j, 16)
                                acc[d, sl] = jnp.maximum(
                                    acc[d, sl], rows[r, sl])
                        return c2
                    lax.fori_loop(0, ng, gbody, 0)
            return carry
        lax.fori_loop(0, (nck + 1) // 2, gpair, 0)

        pltpu.sync_copy(acc.at[pl.ds(0, npt)], agg_hbm.at[pl.ds(n0, npt)])

    return kc


def _tc_dense(m, ka, kx, n, relu, bm, out_pad=None):
    nw = out_pad or n

    def body(a_ref, x_ref, wl_ref, wr_ref, b_ref, o_ref):
        a = a_ref[...]
        a = jnp.where(a == -jnp.inf, jnp.float32(0.0), a)
        r = (jnp.dot(a, wl_ref[...], preferred_element_type=jnp.float32)
             + jnp.dot(x_ref[...], wr_ref[...],
                       preferred_element_type=jnp.float32)
             + b_ref[...])
        if relu:
            r = jnp.maximum(r, jnp.float32(0.0))
        if out_pad is None:
            o_ref[...] = r
        else:
            o_ref[:, pl.ds(0, n)] = r
            o_ref[:, pl.ds(n, out_pad - n)] = jnp.zeros(
                (bm, out_pad - n), jnp.float32)

    grid = (m // bm,)
    return pl.pallas_call(
        body,
        grid=grid,
        in_specs=[
            pl.BlockSpec((bm, ka), lambda i: (i, 0)),
            pl.BlockSpec((bm, kx), lambda i: (i, 0)),
            pl.BlockSpec((ka, n), lambda i: (0, 0)),
            pl.BlockSpec((kx, n), lambda i: (0, 0)),
            pl.BlockSpec((1, n), lambda i: (0, 0)),
        ],
        out_specs=pl.BlockSpec((bm, nw), lambda i: (i, 0)),
        out_shape=jax.ShapeDtypeStruct((m, nw), jnp.float32),
    )


def kernel(x, edge_index, W1l, b1, W1r, W2l, b2, W2r):
    n, d = x.shape
    e = edge_index.shape[1]
    h_dim = W1l.shape[1]
    c_dim = W2l.shape[1]
    npt = (-(-n // NW) + 7) // 8 * 8

    esrc = edge_index[0]
    edst = edge_index[1]
    agg1p, srcl, dstl, cnts = _sc_layer1(n, e, d, npt)(x, esrc, edst)
    agg1 = agg1p[:n]

    hp = _tc_dense(n, d, d, h_dim, True, 2000, out_pad=128)(
        agg1, x, W1l, W1r, b1.reshape(1, -1))

    agg2p = _sc_layer2(n, h_dim, 128, npt)(hp, srcl, dstl, cnts)
    agg2 = agg2p[:n]

    cpad = 8
    w2l = jnp.zeros((h_dim, cpad), jnp.float32).at[:, :c_dim].set(W2l)
    w2rp = jnp.zeros((128, cpad), jnp.float32).at[:h_dim, :c_dim].set(W2r)
    b2p = jnp.zeros((1, cpad), jnp.float32).at[0, :c_dim].set(b2)

    out = _tc_dense(n, h_dim, 128, cpad, False, 2000)(
        agg2, hp, w2l, w2rp, b2p)
    return out[:, :c_dim]


# speedup vs baseline: 4.2361x; 1.1274x over previous
import functools

import jax
import jax.numpy as jnp
from jax import lax
from jax.experimental import pallas as pl
from jax.experimental.pallas import tpu as pltpu
from jax.experimental.pallas import tpu_sc as plsc

NW = 32
CAP = 12800
CE = 8000
CG = 96
CG2 = 128
LPAD = CAP + 512


def _sc_layer1(n_nodes, n_edges, feat, npt):
    npad = NW * npt
    nfv = feat // 16
    mesh = plsc.VectorSubcoreMesh(core_axis_name="c", subcore_axis_name="s")

    @functools.partial(
        pl.kernel,
        out_type=[
            jax.ShapeDtypeStruct((npad, feat), jnp.float32),
            jax.ShapeDtypeStruct((NW, LPAD), jnp.int32),
            jax.ShapeDtypeStruct((NW, LPAD), jnp.int32),
            jax.ShapeDtypeStruct((NW, 16), jnp.int32),
        ],
        mesh=mesh,
        compiler_params=pltpu.CompilerParams(needs_layout_passes=False),
        scratch_types=[
            pltpu.VMEM((npt + 1, feat), jnp.float32),
            pltpu.VMEM((LPAD,), jnp.int32),
            pltpu.VMEM((LPAD,), jnp.int32),
            pltpu.VMEM((CE,), jnp.int32),
            pltpu.VMEM((CE,), jnp.int32),
            pltpu.VMEM((CE,), jnp.int32),
            pltpu.VMEM((CE,), jnp.int32),
            pltpu.VMEM((CG, feat), jnp.float32),
            pltpu.VMEM((CG, feat), jnp.float32),
            pltpu.VMEM((16,), jnp.int32),
            pltpu.SemaphoreType.DMA,
            pltpu.SemaphoreType.DMA,
            pltpu.SemaphoreType.DMA,
            pltpu.SemaphoreType.DMA,
        ],
    )
    def ka(x_hbm, esrc_hbm, edst_hbm, agg_hbm, srcl_hbm, dstl_hbm, cnt_hbm,
           acc, srcl, dstl, srch0, srch1, dsth0, dsth1, rows0, rows1, cstg,
           sem0, sem1, sem2, sem3):
        w = lax.axis_index("s") * 2 + lax.axis_index("c")
        n0 = w * npt

        minf = jnp.full((16,), -jnp.inf, jnp.float32)
        zero16 = jnp.zeros((16,), jnp.int32)

        def initacc(r, carry):
            for j in range(nfv):
                acc[r, pl.ds(16 * j, 16)] = minf
            return carry
        lax.fori_loop(0, npt + 1, initacc, 0)

        def initsrc(i, carry):
            srcl[pl.ds(16 * i, 16)] = zero16
            return carry
        lax.fori_loop(0, LPAD // 16, initsrc, 0)

        nch = n_edges // CE
        sbufs = (srch0, srch1)
        dbufs = (dsth0, dsth1)
        esems = (sem0, sem1)
        dsems = (sem2, sem3)

        pltpu.async_copy(esrc_hbm.at[pl.ds(0, CE)], srch0, sem0)
        pltpu.async_copy(edst_hbm.at[pl.ds(0, CE)], dsth0, sem2)

        def cpair(t, ptr):
            for b in range(2):
                c = 2 * t + b
                nb = 1 - b

                @pl.when(c + 1 < nch)
                def _():
                    e1 = (c + 1) * CE
                    pltpu.async_copy(
                        esrc_hbm.at[pl.ds(e1, CE)], sbufs[nb], esems[nb])
                    pltpu.async_copy(
                        edst_hbm.at[pl.ds(e1, CE)], dbufs[nb], dsems[nb])

                e0 = c * CE
                pltpu.make_async_copy(
                    esrc_hbm.at[pl.ds(e0, CE)], sbufs[b], esems[b]).wait()
                pltpu.make_async_copy(
                    edst_hbm.at[pl.ds(e0, CE)], dbufs[b], dsems[b]).wait()
                srch = sbufs[b]
                dsth = dbufs[b]

                def vec_body(i, p):
                    offs = []
                    svs = []
                    cs = []
                    ms = []
                    for u in range(4):
                        dv = dsth[pl.ds(64 * i + 16 * u, 16)]
                        sv = srch[pl.ds(64 * i + 16 * u, 16)]
                        off = dv - n0
                        m = (off >= 0) & (off < npt)
                        offs.append(off)
                        svs.append(sv)
                        ms.append(m)
                        cs.append(plsc.all_reduce_population_count(m)[0])
                    for u in range(4):
                        plsc.store_compressed(
                            srcl.at[pl.ds(p, 16)], svs[u], mask=ms[u])
                        plsc.store_compressed(
                            dstl.at[pl.ds(p, 16)], offs[u], mask=ms[u])
                        p = jnp.minimum(p + cs[u], jnp.int32(CAP))
                    return p

                ptr = lax.fori_loop(0, CE // 64, vec_body, ptr)
            return ptr

        ptr = lax.fori_loop(0, nch // 2, cpair, jnp.int32(0))
        cnt = ptr

        srcl[pl.ds(cnt, 16)] = zero16
        dstl[pl.ds(cnt, 16)] = jnp.full((16,), npt, jnp.int32)
        cntp = ((cnt + 15) // 16) * 16

        nck = (cntp + CG - 1) // CG
        bufs = (rows0, rows1)
        sems = (sem0, sem1)

        @pl.when(nck > 0)
        def _():
            pltpu.async_copy(x_hbm.at[srcl.at[pl.ds(0, CG)]], rows0, sem0)

        def gpair(t, carry):
            for b in range(2):
                c = 2 * t + b
                nb = (b + 1) % 2

                @pl.when(c < nck)
                def _():
                    base = c * CG

                    @pl.when(c + 1 < nck)
                    def _():
                        pltpu.async_copy(
                            x_hbm.at[srcl.at[pl.ds((c + 1) * CG, CG)]],
                            bufs[nb], sems[nb])

                    pltpu.make_async_copy(
                        x_hbm.at[srcl.at[pl.ds(base, CG)]],
                        bufs[b], sems[b]).wait()
                    rows = bufs[b]
                    ng = jnp.minimum(jnp.int32(CG // 16),
                                     (cntp - base) // 16)

                    def gbody(g, c2):
                        dv = dstl[pl.ds(base + 16 * g, 16)]
                        for lane in range(16):
                            d = dv[lane]
                            r = 16 * g + lane
                            for j in range(nfv):
                                sl = pl.ds(16 * j, 16)
                                acc[d, sl] = jnp.maximum(
                                    acc[d, sl], rows[r, sl])
                        return c2
                    lax.fori_loop(0, ng, gbody, 0)
            return carry
        lax.fori_loop(0, (nck + 1) // 2, gpair, 0)

        pltpu.sync_copy(acc.at[pl.ds(0, npt)], agg_hbm.at[pl.ds(n0, npt)])
        pltpu.sync_copy(srcl, srcl_hbm.at[w])
        pltpu.sync_copy(dstl, dstl_hbm.at[w])
        cstg[...] = jnp.full((16,), cntp, jnp.int32)
        pltpu.sync_copy(cstg, cnt_hbm.at[w])

    return ka


def _sc_layer2(n_nodes, feat, fpad, npt):
    npad = NW * npt
    nfv = feat // 16
    mesh = plsc.VectorSubcoreMesh(core_axis_name="c", subcore_axis_name="s")

    @functools.partial(
        pl.kernel,
        out_type=jax.ShapeDtypeStruct((npad, feat), jnp.float32),
        mesh=mesh,
        compiler_params=pltpu.CompilerParams(needs_layout_passes=False),
        scratch_types=[
            pltpu.VMEM((npt + 1, feat), jnp.float32),
            pltpu.VMEM((LPAD,), jnp.int32),
            pltpu.VMEM((LPAD,), jnp.int32),
            pltpu.VMEM((CG2, fpad), jnp.float32),
            pltpu.VMEM((CG2, fpad), jnp.float32),
            pltpu.VMEM((16,), jnp.int32),
            pltpu.SemaphoreType.DMA,
            pltpu.SemaphoreType.DMA,
        ],
    )
    def kc(h_hbm, srcl_hbm, dstl_hbm, cnt_hbm, agg_hbm,
           acc, srcl, dstl, rows0, rows1, cstg, sem0, sem1):
        w = lax.axis_index("s") * 2 + lax.axis_index("c")
        n0 = w * npt

        pltpu.sync_copy(cnt_hbm.at[w], cstg)
        pltpu.sync_copy(srcl_hbm.at[w], srcl)
        pltpu.sync_copy(dstl_hbm.at[w], dstl)
        cntp = cstg[pl.ds(0, 16)][0]

        minf = jnp.full((16,), -jnp.inf, jnp.float32)

        def initacc(r, carry):
            for j in range(nfv):
                acc[r, pl.ds(16 * j, 16)] = minf
            return carry
        lax.fori_loop(0, npt + 1, initacc, 0)

        nck = (cntp + CG2 - 1) // CG2
        bufs = (rows0, rows1)
        sems = (sem0, sem1)

        @pl.when(nck > 0)
        def _():
            pltpu.async_copy(h_hbm.at[srcl.at[pl.ds(0, CG2)]], rows0, sem0)

        def gpair(t, carry):
            for b in range(2):
                c = 2 * t + b
                nb = (b + 1) % 2

                @pl.when(c < nck)
                def _():
                    base = c * CG2

                    @pl.when(c + 1 < nck)
                    def _():
                        pltpu.async_copy(
                            h_hbm.at[srcl.at[pl.ds((c + 1) * CG2, CG2)]],
                            bufs[nb], sems[nb])

                    pltpu.make_async_copy(
                        h_hbm.at[srcl.at[pl.ds(base, CG2)]],
                        bufs[b], sems[b]).wait()
                    rows = bufs[b]
                    ng = jnp.minimum(jnp.int32(CG2 // 16),
                                     (cntp - base) // 16)

                    def gbody(g, c2):
                        dv = dstl[pl.ds(base + 16 * g, 16)]
                        for lane in range(16):
                            d = dv[lane]
                            r = 16 * g + lane
                            for j in range(nfv):
                                sl = pl.ds(16 * j, 16)
                                acc[d, sl] = jnp.maximum(
                                    acc[d, sl], rows[r, sl])
                        return c2
                    lax.fori_loop(0, ng, gbody, 0)
            return carry
        lax.fori_loop(0, (nck + 1) // 2, gpair, 0)

        pltpu.sync_copy(acc.at[pl.ds(0, npt)], agg_hbm.at[pl.ds(n0, npt)])

    return kc


def _tc_dense(m, ka, kx, n, relu, bm, out_pad=None):
    nw = out_pad or n

    def body(a_ref, x_ref, wl_ref, wr_ref, b_ref, o_ref):
        a = a_ref[...]
        a = jnp.where(a == -jnp.inf, jnp.float32(0.0), a)
        r = (jnp.dot(a, wl_ref[...], preferred_element_type=jnp.float32)
             + jnp.dot(x_ref[...], wr_ref[...],
                       preferred_element_type=jnp.float32)
             + b_ref[...])
        if relu:
            r = jnp.maximum(r, jnp.float32(0.0))
        if out_pad is None:
            o_ref[...] = r
        else:
            o_ref[:, pl.ds(0, n)] = r
            o_ref[:, pl.ds(n, out_pad - n)] = jnp.zeros(
                (bm, out_pad - n), jnp.float32)

    grid = (m // bm,)
    return pl.pallas_call(
        body,
        grid=grid,
        in_specs=[
            pl.BlockSpec((bm, ka), lambda i: (i, 0)),
            pl.BlockSpec((bm, kx), lambda i: (i, 0)),
            pl.BlockSpec((ka, n), lambda i: (0, 0)),
            pl.BlockSpec((kx, n), lambda i: (0, 0)),
            pl.BlockSpec((1, n), lambda i: (0, 0)),
        ],
        out_specs=pl.BlockSpec((bm, nw), lambda i: (i, 0)),
        out_shape=jax.ShapeDtypeStruct((m, nw), jnp.float32),
    )


def kernel(x, edge_index, W1l, b1, W1r, W2l, b2, W2r):
    n, d = x.shape
    e = edge_index.shape[1]
    h_dim = W1l.shape[1]
    c_dim = W2l.shape[1]
    npt = (-(-n // NW) + 7) // 8 * 8

    esrc = edge_index[0]
    edst = edge_index[1]
    agg1p, srcl, dstl, cnts = _sc_layer1(n, e, d, npt)(x, esrc, edst)

    hp = _tc_dense(n, d, d, h_dim, True, 2000, out_pad=128)(
        agg1p, x, W1l, W1r, b1.reshape(1, -1))

    agg2p = _sc_layer2(n, h_dim, 128, npt)(hp, srcl, dstl, cnts)

    cpad = 8
    w2l = jnp.zeros((h_dim, cpad), jnp.float32).at[:, :c_dim].set(W2l)
    w2rp = jnp.zeros((128, cpad), jnp.float32).at[:h_dim, :c_dim].set(W2r)
    b2p = jnp.zeros((1, cpad), jnp.float32).at[0, :c_dim].set(b2)

    out = _tc_dense(n, h_dim, 128, cpad, False, 2000)(
        agg2p, hp, w2l, w2rp, b2p)
    return out[:, :c_dim]
